# Initial kernel scaffold; baseline (speedup 1.0000x reference)
import functools

import jax
import jax.numpy as jnp
from jax import lax
from jax.experimental import pallas as pl
from jax.experimental.pallas import tpu as pltpu
from jax.experimental.pallas import tpu_sc as plsc

N = 10000
E = 320000
D_IN = 128
D_H = 64
N_CLASSES = 16

NC = 2
NS = 16
NW = NC * NS
CHUNK = 128
N_PAD = 10240
EPW = 80
E_PAD = NW * EPW * CHUNK
ROWS_PER_TILE = N_PAD // NS


def _sc_segment_sum(d_feat):
    mesh = plsc.VectorSubcoreMesh(
        core_axis_name="c", subcore_axis_name="s", num_cores=NC,
        num_subcores=NS)
    zero16 = jnp.zeros((16,), jnp.float32)
    n_lane_grps = d_feat // 16

    @functools.partial(
        pl.kernel,
        out_type=jax.ShapeDtypeStruct((NC, N_PAD, d_feat), jnp.float32),
        mesh=mesh,
        scratch_types=[
            pltpu.VMEM((EPW, CHUNK), jnp.int32),
            pltpu.VMEM((EPW, CHUNK), jnp.int32),
            pltpu.VMEM((CHUNK, d_feat), jnp.float32),
            pltpu.VMEM((CHUNK, d_feat), jnp.float32),
            pltpu.VMEM_SHARED((N_PAD, d_feat), jnp.float32),
            pltpu.SemaphoreType.DMA,
            pltpu.SemaphoreType.DMA,
        ],
    )
    def seg_sum(h_hbm, src_hbm, dst_hbm, out_hbm, src_v, dst_v, rows0, rows1,
                acc, sem0, sem1):
        c = lax.axis_index("c")
        s = lax.axis_index("s")
        wid = s * NC + c

        def zrow(j, _):
            for k in range(n_lane_grps):
                rows0[j, pl.ds(16 * k, 16)] = zero16
            return 0
        lax.fori_loop(0, CHUNK, zrow, 0)
        base = s * ROWS_PER_TILE
        for t in range(ROWS_PER_TILE // CHUNK):
            pltpu.sync_copy(rows0, acc.at[pl.ds(base + t * CHUNK, CHUNK)])
        plsc.subcore_barrier()

        pltpu.sync_copy(src_hbm.at[wid], src_v)
        pltpu.sync_copy(dst_hbm.at[wid], dst_v)

        pltpu.async_copy(h_hbm.at[src_v.at[0]], rows0, sem0)

        def body(g, _):
            j = 2 * g
            pltpu.make_async_copy(h_hbm.at[src_v.at[j]], rows0, sem0).wait()
            pltpu.async_copy(h_hbm.at[src_v.at[j + 1]], rows1, sem1)
            pltpu.sync_copy(rows0, acc.at[dst_v.at[j]], add=True)
            pltpu.make_async_copy(
                h_hbm.at[src_v.at[j + 1]], rows1, sem1).wait()

            @pl.when(g < EPW // 2 - 1)
            def _():
                pltpu.async_copy(h_hbm.at[src_v.at[j + 2]], rows0, sem0)

            pltpu.sync_copy(rows1, acc.at[dst_v.at[j + 1]], add=True)
            return 0

        lax.fori_loop(0, EPW // 2, body, 0)
        plsc.subcore_barrier()

        for t in range(ROWS_PER_TILE // CHUNK):
            r = base + t * CHUNK
            pltpu.sync_copy(acc.at[pl.ds(r, CHUNK)], rows0)
            pltpu.sync_copy(rows0, out_hbm.at[c].at[pl.ds(r, CHUNK)])

    return seg_sum


def _input_proj(x, wi_t, bi):
    blk = 1024
    grid = N_PAD // blk

    def body(x_ref, w_ref, b_ref, o_ref):
        h = jnp.dot(x_ref[...], w_ref[...],
                    preferred_element_type=jnp.float32)
        o_ref[...] = jnp.maximum(h + b_ref[...][None, :], 0.0)

    return pl.pallas_call(
        body,
        grid=(grid,),
        in_specs=[
            pl.BlockSpec((blk, D_IN), lambda i: (i, 0)),
            pl.BlockSpec((D_IN, D_H), lambda i: (0, 0)),
            pl.BlockSpec((D_H,), lambda i: (0,)),
        ],
        out_specs=pl.BlockSpec((blk, D_H), lambda i: (i, 0)),
        out_shape=jax.ShapeDtypeStruct((N_PAD, D_H), jnp.float32),
    )(x, wi_t, bi)


def _epilogue(parts, h, wl_t, bl, wr_t, g, b, inv_deg=None, wo_t=None,
              bo=None):
    blk = 1024
    grid = N_PAD // blk
    d_feat = parts.shape[-1]
    first = inv_deg is None
    last = wo_t is not None

    def body(*refs):
        refs = list(refs)
        p0_ref, p1_ref, h_ref = refs[0], refs[1], refs[2]
        i = 3
        if not first:
            invd_ref = refs[i]; i += 1
        wl_ref, bl_ref, wr_ref, g_ref, b_ref = refs[i:i + 5]; i += 5
        if last:
            wo_ref, bo_ref = refs[i], refs[i + 1]; i += 2
        out_ref = refs[i]; i += 1
        if first:
            invd_out = refs[i]

        p = p0_ref[0] + p1_ref[0]
        if first:
            deg = p[:, D_H:D_H + 1]
            inv = 1.0 / jnp.maximum(deg, 1.0)
            invd_out[...] = jnp.broadcast_to(inv, (blk, 8))
            agg = p[:, :D_H] * inv
        else:
            inv = invd_ref[:, 0:1]
            agg = p * inv
        hx = h_ref[...]
        t = (jnp.dot(agg, wl_ref[...], preferred_element_type=jnp.float32)
             + bl_ref[...][None, :]
             + jnp.dot(hx, wr_ref[...], preferred_element_type=jnp.float32))
        mu = jnp.mean(t, axis=-1, keepdims=True)
        var = jnp.mean((t - mu) ** 2, axis=-1, keepdims=True)
        y = (t - mu) * lax.rsqrt(var + 1e-5) * g_ref[...][None, :] \
            + b_ref[...][None, :]
        r = jnp.maximum(y, 0.0) + hx
        if last:
            out_ref[...] = jnp.dot(r, wo_ref[...],
                                   preferred_element_type=jnp.float32) \
                + bo_ref[...][None, :]
        else:
            out_ref[...] = r

    in_specs = [
        pl.BlockSpec((1, blk, d_feat), lambda i: (0, i, 0)),
        pl.BlockSpec((1, blk, d_feat), lambda i: (1, i, 0)),
        pl.BlockSpec((blk, D_H), lambda i: (i, 0)),
    ]
    args = [parts, parts, h]
    if not first:
        in_specs.append(pl.BlockSpec((blk, 8), lambda i: (i, 0)))
        args.append(inv_deg)
    in_specs += [
        pl.BlockSpec((D_H, D_H), lambda i: (0, 0)),
        pl.BlockSpec((D_H,), lambda i: (0,)),
        pl.BlockSpec((D_H, D_H), lambda i: (0, 0)),
        pl.BlockSpec((D_H,), lambda i: (0,)),
        pl.BlockSpec((D_H,), lambda i: (0,)),
    ]
    args += [wl_t, bl, wr_t, g, b]
    if last:
        in_specs += [
            pl.BlockSpec((D_H, N_CLASSES), lambda i: (0, 0)),
            pl.BlockSpec((N_CLASSES,), lambda i: (0,)),
        ]
        args += [wo_t, bo]

    d_out = N_CLASSES if last else D_H
    out_specs = [pl.BlockSpec((blk, d_out), lambda i: (i, 0))]
    out_shape = [jax.ShapeDtypeStruct((N_PAD, d_out), jnp.float32)]
    if first:
        out_specs.append(pl.BlockSpec((blk, 8), lambda i: (i, 0)))
        out_shape.append(jax.ShapeDtypeStruct((N_PAD, 8), jnp.float32))

    res = pl.pallas_call(
        body,
        grid=(grid,),
        in_specs=in_specs,
        out_specs=out_specs,
        out_shape=out_shape,
    )(*args)
    return res


@jax.jit
def kernel(x, edge_index, batch, Wi, bi, Wl1, bl1, Wr1, g1, b1,
           Wl2, bl2, Wr2, g2, b2, Wl3, bl3, Wr3, g3, b3, Wo, bo):
    src = edge_index[0].astype(jnp.int32)
    dst = edge_index[1].astype(jnp.int32)
    pad_e = E_PAD - E
    src_p = jnp.concatenate(
        [src, jnp.zeros((pad_e,), jnp.int32)]).reshape(NW, EPW, CHUNK)
    dst_p = jnp.concatenate(
        [dst, jnp.full((pad_e,), N, jnp.int32)]).reshape(NW, EPW, CHUNK)

    x_p = jnp.pad(x, ((0, N_PAD - N), (0, 0)))

    h0 = _input_proj(x_p, Wi.T, bi)

    h0_aug = jnp.concatenate(
        [h0, jnp.ones((N_PAD, 1), jnp.float32),
         jnp.zeros((N_PAD, 15), jnp.float32)], axis=1)

    parts1 = _sc_segment_sum(D_H + 16)(h0_aug, src_p, dst_p)
    h1, inv_deg = _epilogue(parts1, h0, Wl1.T, bl1, Wr1.T, g1, b1)

    parts2 = _sc_segment_sum(D_H)(h1, src_p, dst_p)
    (h2,) = _epilogue(parts2, h1, Wl2.T, bl2, Wr2.T, g2, b2, inv_deg=inv_deg)

    parts3 = _sc_segment_sum(D_H)(h2, src_p, dst_p)
    (out,) = _epilogue(parts3, h2, Wl3.T, bl3, Wr3.T, g3, b3,
                       inv_deg=inv_deg, wo_t=Wo.T, bo=bo)
    return out[:N]


# speedup vs baseline: 5.3962x; 5.3962x over previous
import functools

import jax
import jax.numpy as jnp
from jax import lax
from jax.experimental import pallas as pl
from jax.experimental.pallas import tpu as pltpu
from jax.experimental.pallas import tpu_sc as plsc

N = 10000
E = 320000
D_IN = 128
D_H = 64
N_CLASSES = 16

NC = 2
NS = 16
NW = NC * NS
CHUNK = 128
N_PAD = 10240
EPW = 80
E_PAD = NW * EPW * CHUNK
ROWS_PER_TILE = N_PAD // NS


def _sc_segment_sum(d_feat):
    mesh = plsc.VectorSubcoreMesh(
        core_axis_name="c", subcore_axis_name="s", num_cores=NC,
        num_subcores=NS)
    n_lane_grps = d_feat // 16

    @functools.partial(
        pl.kernel,
        out_type=jax.ShapeDtypeStruct((NC, N_PAD, d_feat), jnp.float32),
        mesh=mesh,
        scratch_types=[
            pltpu.VMEM((EPW, CHUNK), jnp.int32),
            pltpu.VMEM((EPW, CHUNK), jnp.int32),
            pltpu.VMEM((CHUNK, d_feat), jnp.float32),
            pltpu.VMEM((CHUNK, d_feat), jnp.float32),
            pltpu.VMEM_SHARED((N_PAD, d_feat), jnp.float32),
            pltpu.SemaphoreType.DMA,
            pltpu.SemaphoreType.DMA,
        ],
        compiler_params=pltpu.CompilerParams(use_tc_tiling_on_sc=False),
    )
    def seg_sum(h_hbm, src_hbm, dst_hbm, out_hbm, src_v, dst_v, rows0, rows1,
                acc, sem0, sem1):
        c = lax.axis_index("c")
        s = lax.axis_index("s")
        wid = s * NC + c

        zero16 = jnp.zeros((16,), jnp.float32)

        def zrow(j, _):
            for k in range(n_lane_grps):
                rows0[j, pl.ds(16 * k, 16)] = zero16
            return 0
        lax.fori_loop(0, CHUNK, zrow, 0)
        base = s * ROWS_PER_TILE
        for t in range(ROWS_PER_TILE // CHUNK):
            pltpu.sync_copy(rows0, acc.at[pl.ds(base + t * CHUNK, CHUNK)])
        plsc.subcore_barrier()

        pltpu.sync_copy(src_hbm.at[wid], src_v)
        pltpu.sync_copy(dst_hbm.at[wid], dst_v)

        pltpu.async_copy(h_hbm.at[src_v.at[0]], rows0, sem0)

        def body(g, _):
            j = 2 * g
            pltpu.make_async_copy(h_hbm.at[src_v.at[j]], rows0, sem0).wait()
            pltpu.async_copy(h_hbm.at[src_v.at[j + 1]], rows1, sem1)
            pltpu.sync_copy(rows0, acc.at[dst_v.at[j]], add=True)
            pltpu.make_async_copy(
                h_hbm.at[src_v.at[j + 1]], rows1, sem1).wait()

            @pl.when(g < EPW // 2 - 1)
            def _():
                pltpu.async_copy(h_hbm.at[src_v.at[j + 2]], rows0, sem0)

            pltpu.sync_copy(rows1, acc.at[dst_v.at[j + 1]], add=True)
            return 0

        lax.fori_loop(0, EPW // 2, body, 0)
        plsc.subcore_barrier()

        for t in range(ROWS_PER_TILE // CHUNK):
            r = base + t * CHUNK
            pltpu.sync_copy(acc.at[pl.ds(r, CHUNK)], rows0)
            pltpu.sync_copy(rows0, out_hbm.at[c].at[pl.ds(r, CHUNK)])

    return seg_sum


def _input_proj(x, wi_t, bi):
    blk = 1024
    grid = N_PAD // blk

    def body(x_ref, w_ref, b_ref, o_ref):
        h = jnp.dot(x_ref[...], w_ref[...],
                    preferred_element_type=jnp.float32)
        o_ref[...] = jnp.maximum(h + b_ref[...][None, :], 0.0)

    return pl.pallas_call(
        body,
        grid=(grid,),
        in_specs=[
            pl.BlockSpec((blk, D_IN), lambda i: (i, 0)),
            pl.BlockSpec((D_IN, D_H), lambda i: (0, 0)),
            pl.BlockSpec((D_H,), lambda i: (0,)),
        ],
        out_specs=pl.BlockSpec((blk, D_H), lambda i: (i, 0)),
        out_shape=jax.ShapeDtypeStruct((N_PAD, D_H), jnp.float32),
    )(x, wi_t, bi)


def _epilogue(parts, h, wl_t, bl, wr_t, g, b, inv_deg=None, wo_t=None,
              bo=None):
    blk = 1024
    grid = N_PAD // blk
    d_feat = parts.shape[-1]
    first = inv_deg is None
    last = wo_t is not None

    def body(*refs):
        refs = list(refs)
        p0_ref, p1_ref, h_ref = refs[0], refs[1], refs[2]
        i = 3
        if not first:
            invd_ref = refs[i]; i += 1
        wl_ref, bl_ref, wr_ref, g_ref, b_ref = refs[i:i + 5]; i += 5
        if last:
            wo_ref, bo_ref = refs[i], refs[i + 1]; i += 2
        out_ref = refs[i]; i += 1
        if first:
            invd_out = refs[i]

        p = p0_ref[0] + p1_ref[0]
        if first:
            deg = p[:, D_H:D_H + 1]
            inv = 1.0 / jnp.maximum(deg, 1.0)
            invd_out[...] = jnp.broadcast_to(inv, (blk, 8))
            agg = p[:, :D_H] * inv
        else:
            inv = invd_ref[:, 0:1]
            agg = p * inv
        hx = h_ref[...]
        t = (jnp.dot(agg, wl_ref[...], preferred_element_type=jnp.float32)
             + bl_ref[...][None, :]
             + jnp.dot(hx, wr_ref[...], preferred_element_type=jnp.float32))
        mu = jnp.mean(t, axis=-1, keepdims=True)
        var = jnp.mean((t - mu) ** 2, axis=-1, keepdims=True)
        y = (t - mu) * lax.rsqrt(var + 1e-5) * g_ref[...][None, :] \
            + b_ref[...][None, :]
        r = jnp.maximum(y, 0.0) + hx
        if last:
            out_ref[...] = jnp.dot(r, wo_ref[...],
                                   preferred_element_type=jnp.float32) \
                + bo_ref[...][None, :]
        else:
            out_ref[...] = r

    in_specs = [
        pl.BlockSpec((1, blk, d_feat), lambda i: (0, i, 0)),
        pl.BlockSpec((1, blk, d_feat), lambda i: (1, i, 0)),
        pl.BlockSpec((blk, D_H), lambda i: (i, 0)),
    ]
    args = [parts, parts, h]
    if not first:
        in_specs.append(pl.BlockSpec((blk, 8), lambda i: (i, 0)))
        args.append(inv_deg)
    in_specs += [
        pl.BlockSpec((D_H, D_H), lambda i: (0, 0)),
        pl.BlockSpec((D_H,), lambda i: (0,)),
        pl.BlockSpec((D_H, D_H), lambda i: (0, 0)),
        pl.BlockSpec((D_H,), lambda i: (0,)),
        pl.BlockSpec((D_H,), lambda i: (0,)),
    ]
    args += [wl_t, bl, wr_t, g, b]
    if last:
        in_specs += [
            pl.BlockSpec((D_H, N_CLASSES), lambda i: (0, 0)),
            pl.BlockSpec((N_CLASSES,), lambda i: (0,)),
        ]
        args += [wo_t, bo]

    d_out = N_CLASSES if last else D_H
    out_specs = [pl.BlockSpec((blk, d_out), lambda i: (i, 0))]
    out_shape = [jax.ShapeDtypeStruct((N_PAD, d_out), jnp.float32)]
    if first:
        out_specs.append(pl.BlockSpec((blk, 8), lambda i: (i, 0)))
        out_shape.append(jax.ShapeDtypeStruct((N_PAD, 8), jnp.float32))

    res = pl.pallas_call(
        body,
        grid=(grid,),
        in_specs=in_specs,
        out_specs=out_specs,
        out_shape=out_shape,
    )(*args)
    return res


@jax.jit
def kernel(x, edge_index, batch, Wi, bi, Wl1, bl1, Wr1, g1, b1,
           Wl2, bl2, Wr2, g2, b2, Wl3, bl3, Wr3, g3, b3, Wo, bo):
    src = edge_index[0].astype(jnp.int32)
    dst = edge_index[1].astype(jnp.int32)
    pad_e = E_PAD - E
    src_p = jnp.concatenate(
        [src, jnp.zeros((pad_e,), jnp.int32)]).reshape(NW, EPW, CHUNK)
    dst_p = jnp.concatenate(
        [dst, jnp.full((pad_e,), N, jnp.int32)]).reshape(NW, EPW, CHUNK)

    x_p = jnp.pad(x, ((0, N_PAD - N), (0, 0)))

    h0 = _input_proj(x_p, Wi.T, bi)

    h0_aug = jnp.concatenate(
        [h0, jnp.ones((N_PAD, 1), jnp.float32),
         jnp.zeros((N_PAD, 15), jnp.float32)], axis=1)

    parts1 = _sc_segment_sum(D_H + 16)(h0_aug, src_p, dst_p)
    h1, inv_deg = _epilogue(parts1, h0, Wl1.T, bl1, Wr1.T, g1, b1)

    parts2 = _sc_segment_sum(D_H)(h1, src_p, dst_p)
    (h2,) = _epilogue(parts2, h1, Wl2.T, bl2, Wr2.T, g2, b2, inv_deg=inv_deg)

    parts3 = _sc_segment_sum(D_H)(h2, src_p, dst_p)
    (out,) = _epilogue(parts3, h2, Wl3.T, bl3, Wr3.T, g3, b3,
                       inv_deg=inv_deg, wo_t=Wo.T, bo=bo)
    return out[:N]
